# baseline (speedup 1.0000x reference)
import functools

import jax
import jax.numpy as jnp
from jax import lax
from jax.experimental import pallas as pl
from jax.experimental.pallas import tpu as pltpu
from jax.experimental.pallas import tpu_sc as plsc

B = 16384
F = 26
D = 16
FIELD = 100000
TOTAL = F * FIELD

NW = 32
SPW = B // NW
FPAD = 32
SPC = 128 // FPAD
NCHUNK = SPW // SPC
LROWS_PER_W = NCHUNK
LIN_BLK = 64
LIN_ROWS_PER_W = SPW * F // 128


def _sc_body(line_hbm, sub_hbm, lval_hbm, emb_hbm, cross_hbm, lsum_hbm,
             idx_v, sub_v, linv_v, lines0, lines1, cross_v, lsum_v,
             sem0, sem1):
    c = lax.axis_index("c")
    s = lax.axis_index("s")
    wid = s * 2 + c

    pltpu.sync_copy(line_hbm.at[pl.ds(wid * LROWS_PER_W, LROWS_PER_W)], idx_v)
    pltpu.sync_copy(sub_hbm.at[pl.ds(wid * LROWS_PER_W, LROWS_PER_W)], sub_v)
    pltpu.sync_copy(lval_hbm.at[pl.ds(wid * LIN_ROWS_PER_W, LIN_ROWS_PER_W)],
                    linv_v)

    bufs = (lines0, lines1)
    sems = (sem0, sem1)

    def emb_cp(b, buf):
        return pltpu.make_async_copy(
            emb_hbm.at[idx_v.at[b]], bufs[buf], sems[buf])

    emb_cp(0, 0).start()
    emb_cp(1, 1).start()

    def process(b, buf):
        emb_cp(b, buf).wait()
        lines = bufs[buf]
        for i in range(SPC):
            base = i * FPAD
            sub_lo = sub_v[b, pl.ds(base, 16)]
            sub_hi = sub_v[b, pl.ds(base + 16, 16)]
            r = lines[base, pl.ds(sub_lo[0] * D, D)]
            s_acc = r
            q_acc = r * r
            for f in range(1, F):
                sub = sub_lo[f] if f < 16 else sub_hi[f - 16]
                r = lines[base + f, pl.ds(sub * D, D)]
                s_acc = s_acc + r
                q_acc = q_acc + r * r
            cross_v[b * SPC + i, :] = 0.5 * (s_acc * s_acc - q_acc)

        @pl.when(b + 2 < NCHUNK)
        def _():
            emb_cp(b + 2, buf).start()

    def body(k, carry):
        process(2 * k, 0)
        process(2 * k + 1, 1)
        return carry

    lax.fori_loop(0, NCHUNK // 2, body, 0, unroll=False)

    def lin_body(blk, carry):
        base_row = blk * (LIN_BLK * F // 128)
        for g in range(LIN_BLK // 16):
            o = g * 16
            acc = linv_v[base_row + o // 128, pl.ds(o % 128, 16)]
            for f in range(1, F):
                o = f * LIN_BLK + g * 16
                acc = acc + linv_v[base_row + o // 128, pl.ds(o % 128, 16)]
            lsum_v[pl.ds(blk * LIN_BLK + g * 16, 16)] = acc
        return carry

    lax.fori_loop(0, SPW // LIN_BLK, lin_body, 0, unroll=False)

    pltpu.sync_copy(cross_v, cross_hbm.at[pl.ds(wid * SPW, SPW)])
    pltpu.sync_copy(lsum_v, lsum_hbm.at[pl.ds(wid * SPW, SPW)])


_sc_gather = functools.partial(
    pl.kernel,
    mesh=plsc.VectorSubcoreMesh(core_axis_name="c", subcore_axis_name="s"),
    out_type=[
        jax.ShapeDtypeStruct((B, D), jnp.float32),
        jax.ShapeDtypeStruct((B,), jnp.float32),
    ],
    scratch_types=[
        pltpu.VMEM((LROWS_PER_W, 128), jnp.int32),
        pltpu.VMEM((LROWS_PER_W, 128), jnp.int32),
        pltpu.VMEM((LIN_ROWS_PER_W, 128), jnp.float32),
        pltpu.VMEM((128, 128), jnp.float32),
        pltpu.VMEM((128, 128), jnp.float32),
        pltpu.VMEM((SPW, D), jnp.float32),
        pltpu.VMEM((SPW,), jnp.float32),
        pltpu.SemaphoreType.DMA,
        pltpu.SemaphoreType.DMA,
    ],
    compiler_params=pltpu.CompilerParams(use_tc_tiling_on_sc=False),
)(_sc_body)


def _bn(h, ones_row, g, b, eps=1e-5):
    m = jnp.dot(ones_row, h, preferred_element_type=jnp.float32)
    ms = jnp.dot(ones_row, h * h, preferred_element_type=jnp.float32)
    scale = g * lax.rsqrt(ms - m * m + eps)
    shift = b - m * scale
    return h * scale + shift


def _mlp_body(cross_ref, lsum_ref, g0_ref, b0_ref, w1_ref, b1_ref, g1_ref,
              be1_ref, w2_ref, b2_ref, g2_ref, be2_ref, w3_ref, b3_ref,
              bias_ref, out_ref):
    ones_row = jnp.full((1, B), 1.0 / B, dtype=jnp.float32)
    cross = _bn(cross_ref[...], ones_row, g0_ref[...], b0_ref[...])
    h = jnp.dot(cross, w1_ref[...], preferred_element_type=jnp.float32)
    h = jnp.maximum(_bn(h + b1_ref[...], ones_row, g1_ref[...], be1_ref[...]), 0.0)
    h = jnp.dot(h, w2_ref[...], preferred_element_type=jnp.float32)
    h = jnp.maximum(_bn(h + b2_ref[...], ones_row, g2_ref[...], be2_ref[...]), 0.0)
    mlp = jnp.dot(h, w3_ref[...], preferred_element_type=jnp.float32)
    out_ref[...] = mlp + b3_ref[...] + lsum_ref[...] + bias_ref[...]


def kernel(x, emb_table, lin_table, lin_bias, bn0_gamma, bn0_beta,
           W1, b1, g1, be1, W2, b2, g2, be2, W3, b3):
    offsets = (jnp.arange(F, dtype=x.dtype) * FIELD)[None, :]
    xi = (x + offsets).astype(jnp.int32)
    xi_pad = jnp.concatenate([xi, xi[:, :FPAD - F]], axis=1)
    line_rows = (xi_pad >> 3).reshape(B * FPAD // 128, 128)
    sub_rows = (xi_pad & 7).reshape(B * FPAD // 128, 128)
    emb2 = emb_table.reshape(TOTAL // 8, 128)

    xi_t = (xi.reshape(NW, SPW // LIN_BLK, LIN_BLK, F)
            .transpose(0, 1, 3, 2)
            .reshape(-1))
    lvals = jnp.take(lin_table, xi_t, axis=0, mode="clip")
    lvals = lvals.reshape(B * F // 128, 128)

    cross, lsum = _sc_gather(line_rows, sub_rows, lvals, emb2)

    out = pl.pallas_call(
        _mlp_body,
        out_shape=jax.ShapeDtypeStruct((B, 1), jnp.float32),
    )(
        cross, lsum.reshape(B, 1),
        bn0_gamma.reshape(1, D), bn0_beta.reshape(1, D),
        W1, b1.reshape(1, -1), g1.reshape(1, -1), be1.reshape(1, -1),
        W2, b2.reshape(1, -1), g2.reshape(1, -1), be2.reshape(1, -1),
        W3, b3.reshape(1, 1), lin_bias.reshape(1, 1),
    )
    return out
